# Initial kernel scaffold; baseline (speedup 1.0000x reference)
import jax
import jax.numpy as jnp
from jax.experimental import pallas as pl


def kernel(x, segment_ids, W, b):
    raise NotImplementedError("write your pallas kernel here")


# speedup vs baseline: 2.2611x; 2.2611x over previous
import functools

import jax
import jax.numpy as jnp
from jax import lax
from jax.experimental import pallas as pl
from jax.experimental.pallas import tpu as pltpu
from jax.experimental.pallas import tpu_sc as plsc

N = 100000
D = 128
S = 512
NW = 32
SEGS_PW = S // NW
CHUNK = 512
NF = D // 16


def _body(x_hbm, ids_hbm, w_hbm, b_hbm, bnd_hbm, out_hbm,
          xbuf, idbuf, wbuf, bbuf, bndbuf, acc):
    wid = lax.axis_index("s") * 2 + lax.axis_index("c")

    pltpu.sync_copy(w_hbm, wbuf)
    pltpu.sync_copy(b_hbm, bbuf)
    pltpu.sync_copy(bnd_hbm, bndbuf)

    zero = jnp.zeros((16,), jnp.float32)
    ninf = jnp.full((16,), -jnp.inf, jnp.float32)
    for s_ in range(SEGS_PW):
        for f_ in range(NF):
            acc[s_, pl.ds(16 * f_, 16)] = zero
            acc[s_, pl.ds(D + 16 * f_, 16)] = ninf

    wvecs = [wbuf[pl.ds(16 * f_, 16)] for f_ in range(NF)]
    bvec = bbuf[...]

    bv = bndbuf[pl.ds(wid, 16)]
    start = bv[0]
    end = bv[1]
    seg_base = SEGS_PW * wid
    nchunks = (end - start + CHUNK - 1) // CHUNK

    def chunk_body(c, _):
        row0 = start + c * CHUNK
        cnt = jnp.minimum(CHUNK, end - row0)
        xbase = jnp.minimum(row0, N - CHUNK)
        xoff = row0 - xbase
        abase = (xbase // 8) * 8
        adelta = xbase - abase
        pltpu.sync_copy(x_hbm.at[pl.ds(xbase * D, CHUNK * D)], xbuf)
        pltpu.sync_copy(ids_hbm.at[pl.ds(abase, CHUNK + 8)],
                        idbuf.at[pl.ds(0, CHUNK + 8)])

        def row_body(r, _):
            seg = idbuf[pl.ds(r + adelta, 16)][0] - seg_base
            xv = [xbuf[pl.ds(r * D + 16 * f_, 16)] for f_ in range(NF)]
            p = xv[0] * wvecs[0]
            for f_ in range(1, NF):
                p = p + xv[f_] * wvecs[f_]
            t = jnp.full((16,), jnp.sum(p), jnp.float32) + bvec
            wv = 1.0 / (1.0 + jnp.exp(-t))
            for f_ in range(NF):
                plsc.addupdate(acc.at[seg, pl.ds(16 * f_, 16)], xv[f_] * wv)
                m = acc[seg, pl.ds(D + 16 * f_, 16)]
                acc[seg, pl.ds(D + 16 * f_, 16)] = jnp.maximum(m, xv[f_])
            return 0

        lax.fori_loop(xoff, xoff + cnt, row_body, 0)
        return 0

    lax.fori_loop(0, nchunks, chunk_body, 0)
    pltpu.sync_copy(acc, out_hbm.at[pl.ds(SEGS_PW * wid, SEGS_PW)])


@jax.jit
def _run(x, ids, wvec, bvec, bnd):
    mesh = plsc.VectorSubcoreMesh(core_axis_name="c", subcore_axis_name="s")
    f = pl.kernel(
        _body,
        out_type=jax.ShapeDtypeStruct((S, 2 * D), jnp.float32),
        mesh=mesh,
        compiler_params=pltpu.CompilerParams(needs_layout_passes=False),
        scratch_types=[
            pltpu.VMEM((CHUNK * D,), jnp.float32),
            pltpu.VMEM((CHUNK + 24,), jnp.int32),
            pltpu.VMEM((D,), jnp.float32),
            pltpu.VMEM((16,), jnp.float32),
            pltpu.VMEM((48,), jnp.int32),
            pltpu.VMEM((SEGS_PW, 2 * D), jnp.float32),
        ],
    )
    return f(x, ids, wvec, bvec, bnd)


def kernel(x, segment_ids, W, b):
    ids = segment_ids.astype(jnp.int32)
    bnd = jnp.searchsorted(
        ids, jnp.arange(0, S + 1, SEGS_PW, dtype=jnp.int32)
    ).astype(jnp.int32)
    bnd = jnp.pad(bnd, (0, 48 - (NW + 1)))
    wvec = W.reshape(D).astype(jnp.float32)
    bvec = jnp.full((16,), b[0], jnp.float32)
    return _run(x.reshape(-1), ids, wvec, bvec, bnd)


# speedup vs baseline: 3.7910x; 1.6766x over previous
import functools

import jax
import jax.numpy as jnp
from jax import lax
from jax.experimental import pallas as pl
from jax.experimental.pallas import tpu as pltpu
from jax.experimental.pallas import tpu_sc as plsc

N = 100000
D = 128
S = 512
NW = 32
SEGS_PW = S // NW
CHUNK = 512
NF = D // 16
SB_LEN = 528


def _body(x_hbm, ids_hbm, w_hbm, b_hbm, sb_hbm, out_hbm,
          xbuf, idbuf, wbuf, bbuf, sbbuf, acc):
    wid = lax.axis_index("s") * 2 + lax.axis_index("c")

    pltpu.sync_copy(w_hbm, wbuf)
    pltpu.sync_copy(b_hbm, bbuf)
    pltpu.sync_copy(sb_hbm, sbbuf)

    zero = jnp.zeros((16,), jnp.float32)
    ninf = jnp.full((16,), -jnp.inf, jnp.float32)
    for s_ in range(SEGS_PW):
        for f_ in range(NF):
            acc[s_, pl.ds(16 * f_, 16)] = zero
            acc[s_, pl.ds(D + 16 * f_, 16)] = ninf

    wvecs = [wbuf[pl.ds(16 * f_, 16)] for f_ in range(NF)]
    bvec = bbuf[...]
    lane = lax.iota(jnp.int32, 16)
    perms = [lane ^ k for k in (1, 2, 4, 8)]

    seg_base = SEGS_PW * wid
    start = sbbuf[pl.ds(seg_base, 16)][0]
    end = sbbuf[pl.ds(seg_base + SEGS_PW, 16)][0]
    nchunks = (end - start + CHUNK - 1) // CHUNK

    def chunk_body(c, _):
        row0 = start + c * CHUNK
        cnt = jnp.minimum(CHUNK, end - row0)
        xbase = jnp.minimum(row0, N - CHUNK)
        xoff = row0 - xbase
        abase = (xbase // 8) * 8
        adelta = xbase - abase
        pltpu.sync_copy(x_hbm.at[pl.ds(xbase * D, CHUNK * D)], xbuf)
        pltpu.sync_copy(ids_hbm.at[pl.ds(abase, CHUNK + 8)],
                        idbuf.at[pl.ds(0, CHUNK + 8)])
        s_first = idbuf[pl.ds(xoff + adelta, 16)][0]
        s_last = idbuf[pl.ds(xoff + adelta + cnt - 1, 16)][0]

        def seg_body(s, _):
            sv = sbbuf[pl.ds(s, 16)]
            blo = jnp.maximum(sv[0], row0) - xbase
            bhi = jnp.minimum(sv[1], row0 + cnt) - xbase

            def row_body(r, carry):
                sums, maxs = carry
                xv = [xbuf[pl.ds(r * D + 16 * f_, 16)] for f_ in range(NF)]
                p = xv[0] * wvecs[0]
                for f_ in range(1, NF):
                    p = p + xv[f_] * wvecs[f_]
                for pm in perms:
                    p = p + p.at[pm].get(mode="promise_in_bounds")
                wv = 1.0 / (1.0 + jnp.exp(-(p + bvec)))
                sums = tuple(sums[f_] + xv[f_] * wv for f_ in range(NF))
                maxs = tuple(jnp.maximum(maxs[f_], xv[f_]) for f_ in range(NF))
                return sums, maxs

            init = (tuple(zero for _ in range(NF)),
                    tuple(ninf for _ in range(NF)))
            sums, maxs = lax.fori_loop(blo, bhi, row_body, init)
            sl = s - seg_base
            for f_ in range(NF):
                plsc.addupdate(acc.at[sl, pl.ds(16 * f_, 16)], sums[f_])
                m = acc[sl, pl.ds(D + 16 * f_, 16)]
                acc[sl, pl.ds(D + 16 * f_, 16)] = jnp.maximum(m, maxs[f_])
            return 0

        lax.fori_loop(s_first, s_last + 1, seg_body, 0)
        return 0

    lax.fori_loop(0, nchunks, chunk_body, 0)
    pltpu.sync_copy(acc, out_hbm.at[pl.ds(SEGS_PW * wid, SEGS_PW)])


@jax.jit
def _run(x, ids, wvec, bvec, sb):
    mesh = plsc.VectorSubcoreMesh(core_axis_name="c", subcore_axis_name="s")
    f = pl.kernel(
        _body,
        out_type=jax.ShapeDtypeStruct((S, 2 * D), jnp.float32),
        mesh=mesh,
        compiler_params=pltpu.CompilerParams(needs_layout_passes=False),
        scratch_types=[
            pltpu.VMEM((CHUNK * D,), jnp.float32),
            pltpu.VMEM((CHUNK + 24,), jnp.int32),
            pltpu.VMEM((D,), jnp.float32),
            pltpu.VMEM((16,), jnp.float32),
            pltpu.VMEM((SB_LEN,), jnp.int32),
            pltpu.VMEM((SEGS_PW, 2 * D), jnp.float32),
        ],
    )
    return f(x, ids, wvec, bvec, sb)


def kernel(x, segment_ids, W, b):
    ids = segment_ids.astype(jnp.int32)
    sb = jnp.searchsorted(
        ids, jnp.arange(0, S + 1, dtype=jnp.int32)
    ).astype(jnp.int32)
    sb = jnp.pad(sb, (0, SB_LEN - (S + 1)))
    wvec = W.reshape(D).astype(jnp.float32)
    bvec = jnp.full((16,), b[0], jnp.float32)
    return _run(x.reshape(-1), ids, wvec, bvec, sb)


# speedup vs baseline: 3.8558x; 1.0171x over previous
import functools

import jax
import jax.numpy as jnp
from jax import lax
from jax.experimental import pallas as pl
from jax.experimental.pallas import tpu as pltpu
from jax.experimental.pallas import tpu_sc as plsc

N = 100000
D = 128
S = 512
NW = 32
SEGS_PW = S // NW
CHUNK = 512
NF = D // 16
SB_LEN = 528


def _body(x_hbm, ids_hbm, w_hbm, b_hbm, sb_hbm, out_hbm,
          xbuf, idbuf, wbuf, bbuf, sbbuf, acc):
    wid = lax.axis_index("s") * 2 + lax.axis_index("c")

    pltpu.sync_copy(w_hbm, wbuf)
    pltpu.sync_copy(b_hbm, bbuf)
    pltpu.sync_copy(sb_hbm, sbbuf)

    zero = jnp.zeros((16,), jnp.float32)
    ninf = jnp.full((16,), -jnp.inf, jnp.float32)
    for s_ in range(SEGS_PW):
        for f_ in range(NF):
            acc[s_, pl.ds(16 * f_, 16)] = zero
            acc[s_, pl.ds(D + 16 * f_, 16)] = ninf

    wvecs = [wbuf[pl.ds(16 * f_, 16)] for f_ in range(NF)]
    bvec = bbuf[...]
    lane = lax.iota(jnp.int32, 16)
    perms = [lane ^ k for k in (1, 2, 4, 8)]

    seg_base = SEGS_PW * wid
    start = sbbuf[pl.ds(seg_base, 16)][0]
    end = sbbuf[pl.ds(seg_base + SEGS_PW, 16)][0]
    nchunks = (end - start + CHUNK - 1) // CHUNK

    def chunk_body(c, _):
        row0 = start + c * CHUNK
        cnt = jnp.minimum(CHUNK, end - row0)
        xbase = jnp.minimum(row0, N - CHUNK)
        xoff = row0 - xbase
        abase = (xbase // 8) * 8
        adelta = xbase - abase
        pltpu.sync_copy(x_hbm.at[pl.ds(xbase * D, CHUNK * D)], xbuf)
        pltpu.sync_copy(ids_hbm.at[pl.ds(abase, CHUNK + 8)],
                        idbuf.at[pl.ds(0, CHUNK + 8)])
        s_first = idbuf[pl.ds(xoff + adelta, 16)][0]
        s_last = idbuf[pl.ds(xoff + adelta + cnt - 1, 16)][0]

        def seg_body(s, _):
            sv = sbbuf[pl.ds(s, 16)]
            blo = jnp.maximum(sv[0], row0) - xbase
            bhi = jnp.minimum(sv[1], row0 + cnt) - xbase

            def one_row(r):
                xv = [xbuf[pl.ds(r * D + 16 * f_, 16)] for f_ in range(NF)]
                p = xv[0] * wvecs[0]
                for f_ in range(1, NF):
                    p = p + xv[f_] * wvecs[f_]
                for pm in perms:
                    p = p + p.at[pm].get(mode="promise_in_bounds")
                wv = 1.0 / (1.0 + jnp.exp(-(p + bvec)))
                return xv, wv

            def pair_body(i, carry):
                sums, maxs = carry
                r = blo + 2 * i
                xv0, wv0 = one_row(r)
                xv1, wv1 = one_row(r + 1)
                sums = tuple(sums[f_] + (xv0[f_] * wv0 + xv1[f_] * wv1)
                             for f_ in range(NF))
                maxs = tuple(jnp.maximum(maxs[f_],
                                         jnp.maximum(xv0[f_], xv1[f_]))
                             for f_ in range(NF))
                return sums, maxs

            def row_body(r, carry):
                sums, maxs = carry
                xv, wv = one_row(r)
                sums = tuple(sums[f_] + xv[f_] * wv for f_ in range(NF))
                maxs = tuple(jnp.maximum(maxs[f_], xv[f_]) for f_ in range(NF))
                return sums, maxs

            init = (tuple(zero for _ in range(NF)),
                    tuple(ninf for _ in range(NF)))
            npair = (bhi - blo) // 2
            carry = lax.fori_loop(0, npair, pair_body, init)
            sums, maxs = lax.fori_loop(blo + 2 * npair, bhi, row_body, carry)
            sl = s - seg_base
            for f_ in range(NF):
                plsc.addupdate(acc.at[sl, pl.ds(16 * f_, 16)], sums[f_])
                m = acc[sl, pl.ds(D + 16 * f_, 16)]
                acc[sl, pl.ds(D + 16 * f_, 16)] = jnp.maximum(m, maxs[f_])
            return 0

        lax.fori_loop(s_first, s_last + 1, seg_body, 0)
        return 0

    lax.fori_loop(0, nchunks, chunk_body, 0)
    pltpu.sync_copy(acc, out_hbm.at[pl.ds(SEGS_PW * wid, SEGS_PW)])


@jax.jit
def _run(x, ids, wvec, bvec, sb):
    mesh = plsc.VectorSubcoreMesh(core_axis_name="c", subcore_axis_name="s")
    f = pl.kernel(
        _body,
        out_type=jax.ShapeDtypeStruct((S, 2 * D), jnp.float32),
        mesh=mesh,
        compiler_params=pltpu.CompilerParams(needs_layout_passes=False),
        scratch_types=[
            pltpu.VMEM((CHUNK * D,), jnp.float32),
            pltpu.VMEM((CHUNK + 24,), jnp.int32),
            pltpu.VMEM((D,), jnp.float32),
            pltpu.VMEM((16,), jnp.float32),
            pltpu.VMEM((SB_LEN,), jnp.int32),
            pltpu.VMEM((SEGS_PW, 2 * D), jnp.float32),
        ],
    )
    return f(x, ids, wvec, bvec, sb)


def kernel(x, segment_ids, W, b):
    ids = segment_ids.astype(jnp.int32)
    sb = jnp.searchsorted(
        ids, jnp.arange(0, S + 1, dtype=jnp.int32)
    ).astype(jnp.int32)
    sb = jnp.pad(sb, (0, SB_LEN - (S + 1)))
    wvec = W.reshape(D).astype(jnp.float32)
    bvec = jnp.full((16,), b[0], jnp.float32)
    return _run(x.reshape(-1), ids, wvec, bvec, sb)


# speedup vs baseline: 6.3737x; 1.6530x over previous
import functools

import jax
import jax.numpy as jnp
from jax import lax
from jax.experimental import pallas as pl
from jax.experimental.pallas import tpu as pltpu
from jax.experimental.pallas import tpu_sc as plsc

N = 100000
D = 128
S = 512
NW = 32
SEGS_PW = S // NW
CHUNK = 512
NF = D // 16


def _scalar(v):
    return v[0] if getattr(v, "ndim", 0) else v


def _body(x_hbm, ids_hbm, w_hbm, b_hbm, out_hbm,
          xbuf, idbuf, wbuf, bbuf, probuf, acc, psem):
    wid = lax.axis_index("s") * 2 + lax.axis_index("c")

    pltpu.sync_copy(w_hbm, wbuf)
    pltpu.sync_copy(b_hbm, bbuf)

    zero = jnp.zeros((16,), jnp.float32)
    ninf = jnp.full((16,), -jnp.inf, jnp.float32)
    for s_ in range(SEGS_PW):
        for f_ in range(NF):
            acc[s_, pl.ds(16 * f_, 16)] = zero
            acc[s_, pl.ds(D + 16 * f_, 16)] = ninf

    wvecs = [wbuf[pl.ds(16 * f_, 16)] for f_ in range(NF)]
    bvec = bbuf[...]
    lane = lax.iota(jnp.int32, 16)
    perms = [lane ^ k for k in (1, 2, 4, 8)]

    seg_base = SEGS_PW * wid

    def search(t):
        def it_body(_, lohi):
            lo, hi = lohi
            step = jnp.maximum((hi - lo + 15) // 16, 1)
            pj = lo + lane * step
            idx = jnp.minimum(pj, N - 1)
            pltpu.async_copy(ids_hbm.at[idx], probuf, psem).wait()
            less = (pj < hi) & (probuf[...] < t)
            c = jnp.sum(less.astype(jnp.int32))
            lo2 = jnp.where(c > 0, lo + (c - 1) * step + 1, lo)
            hi2 = jnp.where(c > 0, jnp.minimum(hi, lo + c * step), lo)
            return lo2, hi2
        lo, _ = lax.fori_loop(0, 6, it_body, (jnp.int32(0), jnp.int32(N)))
        return lo

    start = search(seg_base)
    end = search(seg_base + SEGS_PW)
    nchunks = (end - start + CHUNK - 1) // CHUNK

    def chunk_body(c, _):
        row0 = start + c * CHUNK
        cnt = jnp.minimum(CHUNK, end - row0)
        xbase = jnp.minimum(row0, N - CHUNK)
        xoff = row0 - xbase
        abase = (xbase // 8) * 8
        adelta = xbase - abase
        lim = xoff + cnt
        pltpu.sync_copy(x_hbm.at[pl.ds(xbase * D, CHUNK * D)], xbuf)
        pltpu.sync_copy(ids_hbm.at[pl.ds(abase, CHUNK + 8)],
                        idbuf.at[pl.ds(0, CHUNK + 8)])

        def one_row(r):
            xv = [xbuf[pl.ds(r * D + 16 * f_, 16)] for f_ in range(NF)]
            p = xv[0] * wvecs[0]
            for f_ in range(1, NF):
                p = p + xv[f_] * wvecs[f_]
            for pm in perms:
                p = p + p.at[pm].get(mode="promise_in_bounds")
            wv = 1.0 / (1.0 + jnp.exp(-(p + bvec)))
            return xv, wv

        def pair_body(i, carry):
            blo, sums, maxs = carry
            r = blo + 2 * i
            xv0, wv0 = one_row(r)
            xv1, wv1 = one_row(r + 1)
            sums = tuple(sums[f_] + (xv0[f_] * wv0 + xv1[f_] * wv1)
                         for f_ in range(NF))
            maxs = tuple(jnp.maximum(maxs[f_],
                                     jnp.maximum(xv0[f_], xv1[f_]))
                         for f_ in range(NF))
            return blo, sums, maxs

        def row_body(r, carry):
            sums, maxs = carry
            xv, wv = one_row(r)
            sums = tuple(sums[f_] + xv[f_] * wv for f_ in range(NF))
            maxs = tuple(jnp.maximum(maxs[f_], xv[f_]) for f_ in range(NF))
            return sums, maxs

        def run_cond(st):
            pos = st
            return pos < lim

        def run_body(pos):
            cur = idbuf[pl.ds(pos + adelta, 16)][0]

            def sc_cond(st):
                j, found = st
                return (found == 0) & (j < lim)

            def sc_body(st):
                j, _ = st
                m = idbuf[pl.ds(j + adelta, 16)] != cur
                f = _scalar(plsc.all_reduce_ffs(m))
                return (jnp.where(f < 16, j + f, j + 16).astype(jnp.int32),
                        jnp.where(f < 16, jnp.int32(1), jnp.int32(0)))

            e_j, _ = lax.while_loop(sc_cond, sc_body, (pos, jnp.int32(0)))
            e = jnp.minimum(e_j, lim)

            init = (tuple(zero for _ in range(NF)),
                    tuple(ninf for _ in range(NF)))
            npair = (e - pos) // 2
            _, sums, maxs = lax.fori_loop(
                0, npair, pair_body, (pos,) + init)
            sums, maxs = lax.fori_loop(
                pos + 2 * npair, e, row_body, (sums, maxs))

            sl = cur - seg_base
            for f_ in range(NF):
                plsc.addupdate(acc.at[sl, pl.ds(16 * f_, 16)], sums[f_])
                mv = acc[sl, pl.ds(D + 16 * f_, 16)]
                acc[sl, pl.ds(D + 16 * f_, 16)] = jnp.maximum(mv, maxs[f_])
            return e

        lax.while_loop(run_cond, run_body, xoff)
        return 0

    lax.fori_loop(0, nchunks, chunk_body, 0)
    pltpu.sync_copy(acc, out_hbm.at[pl.ds(SEGS_PW * wid, SEGS_PW)])


@jax.jit
def _run(x, ids, wvec, bvec):
    mesh = plsc.VectorSubcoreMesh(core_axis_name="c", subcore_axis_name="s")
    f = pl.kernel(
        _body,
        out_type=jax.ShapeDtypeStruct((S, 2 * D), jnp.float32),
        mesh=mesh,
        compiler_params=pltpu.CompilerParams(needs_layout_passes=False),
        scratch_types=[
            pltpu.VMEM((CHUNK * D,), jnp.float32),
            pltpu.VMEM((CHUNK + 24,), jnp.int32),
            pltpu.VMEM((D,), jnp.float32),
            pltpu.VMEM((16,), jnp.float32),
            pltpu.VMEM((16,), jnp.int32),
            pltpu.VMEM((SEGS_PW, 2 * D), jnp.float32),
            pltpu.SemaphoreType.DMA,
        ],
    )
    return f(x, ids, wvec, bvec)


def kernel(x, segment_ids, W, b):
    ids = segment_ids.astype(jnp.int32)
    wvec = W.reshape(D).astype(jnp.float32)
    bvec = jnp.full((16,), b[0], jnp.float32)
    return _run(x.reshape(-1), ids, wvec, bvec)
